# Initial kernel scaffold; baseline (speedup 1.0000x reference)
import jax
import jax.numpy as jnp
from jax.experimental import pallas as pl


def kernel(x, hard, W, b, codebook):
    raise NotImplementedError("write your pallas kernel here")


# speedup vs baseline: 1.2162x; 1.2162x over previous
import jax
import jax.numpy as jnp
from jax.experimental import pallas as pl
from jax.experimental.pallas import tpu as pltpu


def _vq_kernel(hard_ref, x_ref, g_ref, w_ref, b_ref, cb_ref, q_ref, idx_ref):
    x = x_ref[...]
    w = w_ref[...]
    cb = cb_ref[...]
    g = g_ref[...]

    h = jax.lax.dot_general(x, w, (((1,), (1,)), ((), ())))
    h = h + b_ref[...]

    hn = h / jnp.maximum(jnp.sqrt(jnp.sum(h * h, axis=1, keepdims=True)), 1e-12)
    cbn = cb / jnp.maximum(jnp.sqrt(jnp.sum(cb * cb, axis=1, keepdims=True)), 1e-12)

    logits = jax.lax.dot_general(hn, cbn, (((1,), (1,)), ((), ()))) * 2.0

    idx = jnp.argmax(logits, axis=1).astype(jnp.int32)
    idx_ref[0, 0, :] = idx

    z = (logits + g) * 2.0
    z = z - jnp.max(z, axis=1, keepdims=True)
    e = jnp.exp(z)
    p = e / jnp.sum(e, axis=1, keepdims=True)

    onehot = (jax.lax.broadcasted_iota(jnp.int32, p.shape, 1)
              == idx[:, None]).astype(p.dtype)
    wgt = jnp.where(hard_ref[0, 0] != 0, onehot, p)

    q_ref[...] = jax.lax.dot_general(wgt, cb, (((1,), (0,)), ((), ())))


def kernel(x, hard, W, b, codebook):
    B, T, D = x.shape
    K, C = codebook.shape
    BT = B * T

    rows = 512
    while BT % rows:
        rows //= 2
    n_tiles = BT // rows

    g = jax.random.gumbel(jax.random.key(42), (BT, K), dtype=x.dtype)

    x2 = x.reshape(BT, D)
    b2 = b.reshape(1, C)
    hard_arr = jnp.asarray(hard).astype(jnp.int32).reshape(1, 1)

    q, idx = pl.pallas_call(
        _vq_kernel,
        grid=(n_tiles,),
        in_specs=[
            pl.BlockSpec(memory_space=pltpu.SMEM),
            pl.BlockSpec((rows, D), lambda i: (i, 0)),
            pl.BlockSpec((rows, K), lambda i: (i, 0)),
            pl.BlockSpec((C, D), lambda i: (0, 0)),
            pl.BlockSpec((1, C), lambda i: (0, 0)),
            pl.BlockSpec((K, C), lambda i: (0, 0)),
        ],
        out_specs=[
            pl.BlockSpec((rows, C), lambda i: (i, 0)),
            pl.BlockSpec((1, 1, rows), lambda i: (i, 0, 0)),
        ],
        out_shape=[
            jax.ShapeDtypeStruct((BT, C), x.dtype),
            jax.ShapeDtypeStruct((n_tiles, 1, rows), jnp.int32),
        ],
    )(hard_arr, x2, g, W, b2, codebook)

    return q.reshape(B, T, C), idx.reshape(B, T)


# speedup vs baseline: 1.2368x; 1.0170x over previous
import jax
import jax.numpy as jnp
from jax.experimental import pallas as pl
from jax.experimental.pallas import tpu as pltpu


def _vq_kernel(hard_ref, x_ref, eg_ref, w_ref, b_ref, cb_ref, cba_ref,
               q_ref, idx_ref, cbn_ref):
    C = q_ref.shape[1]

    @pl.when(pl.program_id(0) == 0)
    def _():
        cb = cb_ref[...]
        s = jnp.sum(cb * cb, axis=1, keepdims=True)
        cbn_ref[...] = cb * jax.lax.rsqrt(jnp.maximum(s, 1e-24))

    x = x_ref[...]

    h = jax.lax.dot_general(x, w_ref[...], (((1,), (1,)), ((), ())))
    h = h + b_ref[...]

    s2 = jnp.sum(h * h, axis=1, keepdims=True)
    hn4 = h * (4.0 * jax.lax.rsqrt(jnp.maximum(s2, 1e-24)))
    l4 = jax.lax.dot_general(hn4, cbn_ref[...], (((1,), (1,)), ((), ())))

    idx = jnp.argmax(l4, axis=1).astype(jnp.int32)
    idx_ref[0, 0, :] = idx

    hard = hard_ref[0, 0]

    @pl.when(hard == 0)
    def _():
        e = jnp.exp(l4) * eg_ref[...]
        out = jax.lax.dot_general(e, cba_ref[...], (((1,), (0,)), ((), ())))
        q_ref[...] = out[:, :C] * (1.0 / out[:, C:C + 1])

    @pl.when(hard != 0)
    def _():
        onehot = (jax.lax.broadcasted_iota(jnp.int32, l4.shape, 1)
                  == idx[:, None]).astype(l4.dtype)
        out = jax.lax.dot_general(onehot, cba_ref[...], (((1,), (0,)), ((), ())))
        q_ref[...] = out[:, :C]


def kernel(x, hard, W, b, codebook):
    B, T, D = x.shape
    K, C = codebook.shape
    BT = B * T

    rows = 512
    while BT % rows:
        rows //= 2
    n_tiles = BT // rows

    g = jax.random.gumbel(jax.random.key(42), (BT, K), dtype=x.dtype)
    eg = jnp.exp(2.0 * g)

    Cp = 128 * ((C + 1 + 127) // 128)
    cba = (jnp.zeros((K, Cp), x.dtype)
           .at[:, :C].set(codebook)
           .at[:, C].set(1.0))

    x2 = x.reshape(BT, D)
    b2 = b.reshape(1, C)
    hard_arr = jnp.asarray(hard).astype(jnp.int32).reshape(1, 1)

    q, idx = pl.pallas_call(
        _vq_kernel,
        grid=(n_tiles,),
        in_specs=[
            pl.BlockSpec(memory_space=pltpu.SMEM),
            pl.BlockSpec((rows, D), lambda i: (i, 0)),
            pl.BlockSpec((rows, K), lambda i: (i, 0)),
            pl.BlockSpec((C, D), lambda i: (0, 0)),
            pl.BlockSpec((1, C), lambda i: (0, 0)),
            pl.BlockSpec((K, C), lambda i: (0, 0)),
            pl.BlockSpec((K, Cp), lambda i: (0, 0)),
        ],
        out_specs=[
            pl.BlockSpec((rows, C), lambda i: (i, 0)),
            pl.BlockSpec((1, 1, rows), lambda i: (i, 0, 0)),
        ],
        out_shape=[
            jax.ShapeDtypeStruct((BT, C), x.dtype),
            jax.ShapeDtypeStruct((n_tiles, 1, rows), jnp.int32),
        ],
        scratch_shapes=[pltpu.VMEM((K, C), x.dtype)],
    )(hard_arr, x2, eg, W, b2, codebook, cba)

    return q.reshape(B, T, C), idx.reshape(B, T)
